# baseline (speedup 1.0000x reference)
import jax
import jax.numpy as jnp
from jax import lax
from jax.experimental import pallas as pl
from jax.experimental.pallas import tpu as pltpu

B, S, D = 16, 4096, 1024
M = 2
CHUNK = 256
NPER = S // CHUNK
TMAX = B * NPER
NBUF = 4
NEG = -1e30
POS = 1e30
MINIT = -1e20


def _pool_body(lens_ref, bs_ref, js_ref, lasts_ref, total_ref,
               G_ref, c0_ref, bq_ref, tokens_hbm,
               trad_ref, learn_ref,
               buf, tail_buf, clf_buf, sum_acc, max_acc, min_acc,
               att_acc, m_acc, l_acc, sems):
    total = total_ref[0]

    def _start(i):
        pltpu.make_async_copy(
            tokens_hbm.at[bs_ref[i], pl.ds(js_ref[i] * CHUNK, CHUNK), :],
            buf.at[lax.rem(i, NBUF)], sems.at[lax.rem(i, NBUF)]).start()

    for k in range(NBUF):
        @pl.when(k < total)
        def _prime(k=k):
            pltpu.make_async_copy(
                tokens_hbm.at[bs_ref[k], pl.ds(js_ref[k] * CHUNK, CHUNK), :],
                buf.at[k], sems.at[k]).start()

    def _reinit():
        sum_acc[...] = jnp.zeros_like(sum_acc)
        max_acc[...] = jnp.full_like(max_acc, NEG)
        min_acc[...] = jnp.full_like(min_acc, POS)
        att_acc[...] = jnp.zeros_like(att_acc)
        m_acc[...] = jnp.full_like(m_acc, MINIT)
        l_acc[...] = jnp.zeros_like(l_acc)

    _reinit()

    def _chunk(i, k):
        bb = bs_ref[i]
        j = js_ref[i]
        L = lens_ref[bb]

        @pl.when((j == 0) & (L >= S))
        def _tail_start():
            pltpu.make_async_copy(
                tokens_hbm.at[bb, pl.ds(S, 1), :], tail_buf,
                sems.at[NBUF]).start()

        pltpu.make_async_copy(
            tokens_hbm.at[bb, pl.ds(j * CHUNK, CHUNK), :],
            buf.at[k], sems.at[k]).wait()

        x = buf[k]
        g = j * CHUNK + lax.broadcasted_iota(jnp.int32, (CHUNK, 1), 0)
        rmask = (g >= 1) & (g <= L)
        ones = jnp.ones((1, CHUNK), jnp.float32)
        full = (j >= 1) & (L >= (j + 1) * CHUNK - 1)

        @pl.when(j == 0)
        def _grab_clf():
            clf_buf[...] = x[0:1, :]

        def _attention(xin, xz, rmaskh):
            xg = lax.dot_general(xin, G_ref[...], (((1,), (0,)), ((), ())),
                                 preferred_element_type=jnp.float32)
            sq = lax.dot_general(xin * xin, G_ref[...],
                                 (((1,), (0,)), ((), ())),
                                 preferred_element_type=jnp.float32)
            mu = xg[:, M:M + 1]
            var = sq[:, M:M + 1] - mu * mu
            rsq = lax.rsqrt(var + 1e-5)
            st = rsq * (xg[:, 0:M] - mu * c0_ref[...]) + bq_ref[...]
            st = jnp.where(rmaskh, st, NEG)
            cmax = jnp.max(st, axis=0, keepdims=True)
            new_m = jnp.maximum(m_acc[...], cmax)
            alpha = jnp.exp(m_acc[...] - new_m)
            p = jnp.exp(st - new_m)
            l_acc[...] = (l_acc[...] * alpha
                          + jnp.sum(p, axis=0, keepdims=True))
            att_acc[...] = (att_acc[...] * alpha.reshape(M, 1)
                            + lax.dot_general(
                                p, xz, (((0,), (0,)), ((), ())),
                                preferred_element_type=jnp.float32))
            m_acc[...] = new_m

        @pl.when(full)
        def _full():
            sum_acc[...] += lax.dot_general(
                ones, x, (((1,), (0,)), ((), ())),
                preferred_element_type=jnp.float32)
            max_acc[...] = jnp.maximum(max_acc[...],
                                       jnp.max(x, axis=0, keepdims=True))
            min_acc[...] = jnp.minimum(min_acc[...],
                                       jnp.min(x, axis=0, keepdims=True))
            _attention(x, x, rmask)

        @pl.when(jnp.logical_not(full))
        def _partial():
            xz = jnp.where(rmask, x, 0.0)
            sum_acc[...] += lax.dot_general(
                ones, xz, (((1,), (0,)), ((), ())),
                preferred_element_type=jnp.float32)
            max_acc[...] = jnp.maximum(
                max_acc[...],
                jnp.max(jnp.where(rmask, x, NEG), axis=0, keepdims=True))
            min_acc[...] = jnp.minimum(
                min_acc[...],
                jnp.min(jnp.where(rmask, x, POS), axis=0, keepdims=True))
            _attention(x, xz, rmask)

        @pl.when(lasts_ref[i] == 1)
        def _finalize():
            @pl.when(L >= S)
            def _merge_tail():
                pltpu.make_async_copy(
                    tokens_hbm.at[bb, pl.ds(S, 1), :], tail_buf,
                    sems.at[NBUF]).wait()
                xt = tail_buf[...]
                sum_acc[...] += xt
                max_acc[...] = jnp.maximum(max_acc[...], xt)
                min_acc[...] = jnp.minimum(min_acc[...], xt)
                _attention(xt, xt, jnp.full((1, 1), True))

            trad_ref[pl.ds(bb, 1), 0:D] = sum_acc[...] / L.astype(jnp.float32)
            trad_ref[pl.ds(bb, 1), D:2 * D] = max_acc[...]
            trad_ref[pl.ds(bb, 1), 2 * D:3 * D] = min_acc[...]
            pmp = att_acc[...] / l_acc[...].reshape(M, 1)
            learn_ref[pl.ds(bb, 1), 0:D] = pmp[0:1, :]
            learn_ref[pl.ds(bb, 1), D:2 * D] = pmp[1:2, :]
            learn_ref[pl.ds(bb, 1), 2 * D:3 * D] = clf_buf[...]
            _reinit()

        @pl.when(i + NBUF < total)
        def _refill():
            i2 = i + NBUF
            pltpu.make_async_copy(
                tokens_hbm.at[bs_ref[i2], pl.ds(js_ref[i2] * CHUNK, CHUNK), :],
                buf.at[k], sems.at[k]).start()

    def outer(it, _):
        base = it * NBUF
        for k in range(NBUF):
            @pl.when(base + k < total)
            def _do(k=k):
                _chunk(base + k, k)
        return None

    nouter = lax.div(total + NBUF - 1, NBUF)
    lax.fori_loop(0, nouter, outer, None)


def _gelu_exact(x):
    return x * 0.5 * (1.0 + lax.erf(x * (2.0 ** -0.5)))


def _mlp_body(x1_ref, x2_ref, w11_ref, b11_ref, w12_ref, b12_ref,
              w21_ref, b21_ref, w22_ref, b22_ref, out_ref):
    h1 = _gelu_exact(
        jnp.dot(x1_ref[...], w11_ref[...],
                preferred_element_type=jnp.float32) + b11_ref[...])
    out_ref[:, 0:D] = jnp.dot(
        h1, w12_ref[...], preferred_element_type=jnp.float32) + b12_ref[...]
    h2 = _gelu_exact(
        jnp.dot(x2_ref[...], w21_ref[...],
                preferred_element_type=jnp.float32) + b21_ref[...])
    out_ref[:, D:2 * D] = jnp.dot(
        h2, w22_ref[...], preferred_element_type=jnp.float32) + b22_ref[...]


@jax.jit
def kernel(tokens, lengths, queries, ln_gamma, ln_beta,
           mlp1_W1, mlp1_b1, mlp1_W2, mlp1_b2,
           mlp2_W1, mlp2_b1, mlp2_W2, mlp2_b2):
    lengths = lengths.astype(jnp.int32)
    qg = (queries * ln_gamma[None, :]).T * (D ** -0.5)
    G = jnp.concatenate(
        [qg, jnp.full((D, 1), 1.0 / D, jnp.float32)], axis=1)
    c0 = jnp.sum(qg, axis=0).reshape(1, M)
    bq = (queries @ ln_beta).reshape(1, M) * (D ** -0.5)

    nblk = jnp.minimum((lengths + CHUNK) // CHUNK, NPER)
    total = jnp.sum(nblk)
    ends = jnp.cumsum(nblk)
    starts = ends - nblk
    ar = jnp.arange(TMAX, dtype=jnp.int32)
    bs = jnp.sum((ar[:, None] >= ends[None, :]).astype(jnp.int32), axis=1)
    bs = jnp.minimum(bs, B - 1)
    js = ar - starts[bs]
    lasts = (ar == (ends[bs] - 1)).astype(jnp.int32)
    total = total.astype(jnp.int32).reshape(1)

    trad, learn = pl.pallas_call(
        _pool_body,
        in_specs=[
            pl.BlockSpec(memory_space=pltpu.SMEM),
            pl.BlockSpec(memory_space=pltpu.SMEM),
            pl.BlockSpec(memory_space=pltpu.SMEM),
            pl.BlockSpec(memory_space=pltpu.SMEM),
            pl.BlockSpec(memory_space=pltpu.SMEM),
            pl.BlockSpec(memory_space=pltpu.VMEM),
            pl.BlockSpec(memory_space=pltpu.VMEM),
            pl.BlockSpec(memory_space=pltpu.VMEM),
            pl.BlockSpec(memory_space=pltpu.MemorySpace.HBM),
        ],
        out_specs=[
            pl.BlockSpec(memory_space=pltpu.VMEM),
            pl.BlockSpec(memory_space=pltpu.VMEM),
        ],
        out_shape=[
            jax.ShapeDtypeStruct((B, 3 * D), jnp.float32),
            jax.ShapeDtypeStruct((B, 3 * D), jnp.float32),
        ],
        scratch_shapes=[
            pltpu.VMEM((NBUF, CHUNK, D), jnp.float32),
            pltpu.VMEM((1, D), jnp.float32),
            pltpu.VMEM((1, D), jnp.float32),
            pltpu.VMEM((1, D), jnp.float32),
            pltpu.VMEM((1, D), jnp.float32),
            pltpu.VMEM((1, D), jnp.float32),
            pltpu.VMEM((M, D), jnp.float32),
            pltpu.VMEM((1, M), jnp.float32),
            pltpu.VMEM((1, M), jnp.float32),
            pltpu.SemaphoreType.DMA((NBUF + 1,)),
        ],
    )(lengths, bs, js, lasts, total, G, c0, bq, tokens)

    out = pl.pallas_call(
        _mlp_body,
        out_shape=jax.ShapeDtypeStruct((B, 2 * D), jnp.float32),
    )(trad, learn,
      mlp1_W1, mlp1_b1.reshape(1, D), mlp1_W2, mlp1_b2.reshape(1, D),
      mlp2_W1, mlp2_b1.reshape(1, D), mlp2_W2, mlp2_b2.reshape(1, D))
    return out


# speedup vs baseline: 1.0198x; 1.0198x over previous
import jax
import jax.numpy as jnp
from jax import lax
from jax.experimental import pallas as pl
from jax.experimental.pallas import tpu as pltpu

B, S, D = 16, 4096, 1024
DH = D // 2
M = 2
CHUNK = 512
NBLK = S // CHUNK + 1
NEG = -1e30
POS = 1e30
MINIT = -1e20


def _pool_body(lens_ref, GL_ref, GR_ref, c0_ref, bq_ref, tokL_ref, tokR_ref,
               trad_ref, learn_ref,
               clf_buf, sum_acc, max_acc, min_acc, att_acc, m_acc, l_acc):
    b = pl.program_id(0)
    j = pl.program_id(1)
    L = lens_ref[b]
    jlast = lax.div(L + CHUNK, CHUNK) - 1

    @pl.when(j == 0)
    def _init():
        sum_acc[...] = jnp.zeros_like(sum_acc)
        max_acc[...] = jnp.full_like(max_acc, NEG)
        min_acc[...] = jnp.full_like(min_acc, POS)
        att_acc[...] = jnp.zeros_like(att_acc)
        m_acc[...] = jnp.full_like(m_acc, MINIT)
        l_acc[...] = jnp.zeros_like(l_acc)
        clf_buf[0:1, 0:DH] = tokL_ref[0, 0:1, :]
        clf_buf[0:1, DH:D] = tokR_ref[0, 0:1, :]

    @pl.when(j <= jlast)
    def _accumulate():
        xl = tokL_ref[0]
        xr = tokR_ref[0]
        g = j * CHUNK + lax.broadcasted_iota(jnp.int32, (CHUNK, 1), 0)
        rmask = (g >= 1) & (g <= L)
        ones = jnp.ones((1, CHUNK), jnp.float32)
        full = (j >= 1) & (L >= (j + 1) * CHUNK - 1)

        def _attention(xla, xra, xzl, xzr, rmaskh):
            xg = (lax.dot_general(xla, GL_ref[...], (((1,), (0,)), ((), ())),
                                  preferred_element_type=jnp.float32)
                  + lax.dot_general(xra, GR_ref[...],
                                    (((1,), (0,)), ((), ())),
                                    preferred_element_type=jnp.float32))
            sq = (lax.dot_general(xla * xla, GL_ref[...],
                                  (((1,), (0,)), ((), ())),
                                  preferred_element_type=jnp.float32)
                  + lax.dot_general(xra * xra, GR_ref[...],
                                    (((1,), (0,)), ((), ())),
                                    preferred_element_type=jnp.float32))
            mu = xg[:, M:M + 1]
            var = sq[:, M:M + 1] - mu * mu
            rsq = lax.rsqrt(var + 1e-5)
            st = rsq * (xg[:, 0:M] - mu * c0_ref[...]) + bq_ref[...]
            st = jnp.where(rmaskh, st, NEG)
            cmax = jnp.max(st, axis=0, keepdims=True)
            new_m = jnp.maximum(m_acc[...], cmax)
            alpha = jnp.exp(m_acc[...] - new_m)
            p = jnp.exp(st - new_m)
            l_acc[...] = (l_acc[...] * alpha
                          + jnp.sum(p, axis=0, keepdims=True))
            a2 = alpha.reshape(M, 1)
            att_acc[:, 0:DH] = (att_acc[:, 0:DH] * a2
                                + lax.dot_general(
                                    p, xzl, (((0,), (0,)), ((), ())),
                                    preferred_element_type=jnp.float32))
            att_acc[:, DH:D] = (att_acc[:, DH:D] * a2
                                + lax.dot_general(
                                    p, xzr, (((0,), (0,)), ((), ())),
                                    preferred_element_type=jnp.float32))
            m_acc[...] = new_m

        @pl.when(full)
        def _full():
            sum_acc[:, 0:DH] += lax.dot_general(
                ones, xl, (((1,), (0,)), ((), ())),
                preferred_element_type=jnp.float32)
            sum_acc[:, DH:D] += lax.dot_general(
                ones, xr, (((1,), (0,)), ((), ())),
                preferred_element_type=jnp.float32)
            max_acc[:, 0:DH] = jnp.maximum(
                max_acc[:, 0:DH], jnp.max(xl, axis=0, keepdims=True))
            max_acc[:, DH:D] = jnp.maximum(
                max_acc[:, DH:D], jnp.max(xr, axis=0, keepdims=True))
            min_acc[:, 0:DH] = jnp.minimum(
                min_acc[:, 0:DH], jnp.min(xl, axis=0, keepdims=True))
            min_acc[:, DH:D] = jnp.minimum(
                min_acc[:, DH:D], jnp.min(xr, axis=0, keepdims=True))
            _attention(xl, xr, xl, xr, rmask)

        @pl.when(jnp.logical_not(full))
        def _partial():
            xzl = jnp.where(rmask, xl, 0.0)
            xzr = jnp.where(rmask, xr, 0.0)
            sum_acc[:, 0:DH] += lax.dot_general(
                ones, xzl, (((1,), (0,)), ((), ())),
                preferred_element_type=jnp.float32)
            sum_acc[:, DH:D] += lax.dot_general(
                ones, xzr, (((1,), (0,)), ((), ())),
                preferred_element_type=jnp.float32)
            max_acc[:, 0:DH] = jnp.maximum(
                max_acc[:, 0:DH],
                jnp.max(jnp.where(rmask, xl, NEG), axis=0, keepdims=True))
            max_acc[:, DH:D] = jnp.maximum(
                max_acc[:, DH:D],
                jnp.max(jnp.where(rmask, xr, NEG), axis=0, keepdims=True))
            min_acc[:, 0:DH] = jnp.minimum(
                min_acc[:, 0:DH],
                jnp.min(jnp.where(rmask, xl, POS), axis=0, keepdims=True))
            min_acc[:, DH:D] = jnp.minimum(
                min_acc[:, DH:D],
                jnp.min(jnp.where(rmask, xr, POS), axis=0, keepdims=True))
            _attention(xl, xr, xzl, xzr, rmask)

    @pl.when(j == NBLK - 1)
    def _finalize():
        trad_ref[0, 0:1, 0:D] = sum_acc[...] / L.astype(jnp.float32)
        trad_ref[0, 0:1, D:2 * D] = max_acc[...]
        trad_ref[0, 0:1, 2 * D:3 * D] = min_acc[...]
        pmp = att_acc[...] / l_acc[...].reshape(M, 1)
        learn_ref[0, 0:1, 0:D] = pmp[0:1, :]
        learn_ref[0, 0:1, D:2 * D] = pmp[1:2, :]
        learn_ref[0, 0:1, 2 * D:3 * D] = clf_buf[...]


def _gelu_exact(x):
    return x * 0.5 * (1.0 + lax.erf(x * (2.0 ** -0.5)))


def _mlp_body(x1_ref, x2_ref, w11_ref, b11_ref, w12_ref, b12_ref,
              w21_ref, b21_ref, w22_ref, b22_ref, out_ref):
    h1 = _gelu_exact(
        jnp.dot(x1_ref[...], w11_ref[...],
                preferred_element_type=jnp.float32) + b11_ref[...])
    out_ref[:, 0:D] = jnp.dot(
        h1, w12_ref[...], preferred_element_type=jnp.float32) + b12_ref[...]
    h2 = _gelu_exact(
        jnp.dot(x2_ref[...], w21_ref[...],
                preferred_element_type=jnp.float32) + b21_ref[...])
    out_ref[:, D:2 * D] = jnp.dot(
        h2, w22_ref[...], preferred_element_type=jnp.float32) + b22_ref[...]


def _tok_index_l(b, j, lens):
    jl = lax.div(lens[b] + CHUNK, CHUNK) - 1
    return (b, jnp.minimum(j, jl), 0)


def _tok_index_r(b, j, lens):
    jl = lax.div(lens[b] + CHUNK, CHUNK) - 1
    return (b, jnp.minimum(j, jl), 1)


@jax.jit
def kernel(tokens, lengths, queries, ln_gamma, ln_beta,
           mlp1_W1, mlp1_b1, mlp1_W2, mlp1_b2,
           mlp2_W1, mlp2_b1, mlp2_W2, mlp2_b2):
    lengths = lengths.astype(jnp.int32)
    qg = (queries * ln_gamma[None, :]).T * (D ** -0.5)
    G = jnp.concatenate(
        [qg, jnp.full((D, 1), 1.0 / D, jnp.float32)], axis=1)
    c0 = jnp.sum(qg, axis=0).reshape(1, M)
    bq = (queries @ ln_beta).reshape(1, M) * (D ** -0.5)

    grid_spec = pltpu.PrefetchScalarGridSpec(
        num_scalar_prefetch=1,
        grid=(B, NBLK),
        in_specs=[
            pl.BlockSpec(memory_space=pltpu.VMEM),
            pl.BlockSpec(memory_space=pltpu.VMEM),
            pl.BlockSpec(memory_space=pltpu.VMEM),
            pl.BlockSpec(memory_space=pltpu.VMEM),
            pl.BlockSpec((1, CHUNK, DH), _tok_index_l),
            pl.BlockSpec((1, CHUNK, DH), _tok_index_r),
        ],
        out_specs=[
            pl.BlockSpec((1, 1, 3 * D), lambda b, j, lens: (b, 0, 0)),
            pl.BlockSpec((1, 1, 3 * D), lambda b, j, lens: (b, 0, 0)),
        ],
        scratch_shapes=[
            pltpu.VMEM((1, D), jnp.float32),
            pltpu.VMEM((1, D), jnp.float32),
            pltpu.VMEM((1, D), jnp.float32),
            pltpu.VMEM((1, D), jnp.float32),
            pltpu.VMEM((M, D), jnp.float32),
            pltpu.VMEM((1, M), jnp.float32),
            pltpu.VMEM((1, M), jnp.float32),
        ],
    )
    trad, learn = pl.pallas_call(
        _pool_body,
        grid_spec=grid_spec,
        out_shape=[
            jax.ShapeDtypeStruct((B, 1, 3 * D), jnp.float32),
            jax.ShapeDtypeStruct((B, 1, 3 * D), jnp.float32),
        ],
        compiler_params=pltpu.CompilerParams(
            dimension_semantics=("arbitrary", "arbitrary")),
    )(lengths, G[0:DH], G[DH:D], c0, bq, tokens, tokens)

    out = pl.pallas_call(
        _mlp_body,
        out_shape=jax.ShapeDtypeStruct((B, 2 * D), jnp.float32),
    )(trad.reshape(B, 3 * D), learn.reshape(B, 3 * D),
      mlp1_W1, mlp1_b1.reshape(1, D), mlp1_W2, mlp1_b2.reshape(1, D),
      mlp2_W1, mlp2_b1.reshape(1, D), mlp2_W2, mlp2_b2.reshape(1, D))
    return out
